# baseline (speedup 1.0000x reference)
import jax
import jax.numpy as jnp
from jax.experimental import pallas as pl
from jax.experimental.pallas import tpu as pltpu


def _main_kernel(a1_ref, a2_ref, x_ref, wt_ref, b_ref, o_ref, y_ref):
    @pl.when(pl.program_id(0) == 0)
    def _():
        y_ref[...] = jnp.dot(x_ref[...], wt_ref[...],
                             preferred_element_type=jnp.float32
                             ).astype(jnp.bfloat16)

    y = y_ref[...]
    b_vec = b_ref[...]
    acc1 = jnp.dot(a1_ref[0].astype(jnp.bfloat16), y,
                   preferred_element_type=jnp.float32)
    o_ref[0] = jnp.maximum(acc1 + b_vec, 0.0)
    acc2 = jnp.dot(a2_ref[0].astype(jnp.bfloat16), y,
                   preferred_element_type=jnp.float32)
    o_ref[1] = jnp.maximum(acc2 + b_vec, 0.0)


def kernel(node_features, adjacency_matrix, W, b):
    n, d_in = node_features.shape
    d_out = W.shape[0]
    wt = W.T
    b2d = b.reshape(1, d_out)
    h = n // 2
    a3 = adjacency_matrix.reshape(2, h, n)

    bm = 200
    out = pl.pallas_call(
        _main_kernel,
        grid=(h // bm,),
        in_specs=[
            pl.BlockSpec((1, bm, n), lambda i: (0, i, 0)),
            pl.BlockSpec((1, bm, n), lambda i: (1, i, 0)),
            pl.BlockSpec((n, d_in), lambda i: (0, 0)),
            pl.BlockSpec((d_in, d_out), lambda i: (0, 0)),
            pl.BlockSpec((1, d_out), lambda i: (0, 0)),
        ],
        out_specs=pl.BlockSpec((2, bm, d_out), lambda i: (0, i, 0)),
        out_shape=jax.ShapeDtypeStruct((2, h, d_out), jnp.float32),
        scratch_shapes=[pltpu.VMEM((n, d_out), jnp.bfloat16)],
    )(a3, a3, node_features, wt, b2d)
    return out.reshape(n, d_out)
